# Initial kernel scaffold; baseline (speedup 1.0000x reference)
import jax
import jax.numpy as jnp
from jax.experimental import pallas as pl


def kernel(baseline_weight, delta_logit_weight, inside, coordinates, local_region_ix, local_cell_ix, labels, regions_oi):
    raise NotImplementedError("write your pallas kernel here")


# speedup vs baseline: 6.1699x; 6.1699x over previous
import functools
import math

import jax
import jax.numpy as jnp
from jax import lax
from jax.experimental import pallas as pl
from jax.experimental.pallas import tpu as pltpu
from jax.experimental.pallas import tpu_sc as plsc

BINSIZE = 200
WIDTH = 20000
BINWIDTH = 100
N_REGIONS = 50000
N_CLUSTERS = 8
N_FRAG = 1000000
N_CELLS = 4096
N_REGIONS_OI = 1024

NC, NS = 2, 16
NW = NC * NS

TILE = 2000
NT = N_FRAG // TILE
VECS = TILE // 16

HEIGHTS_FLAT = N_REGIONS_OI * N_CLUSTERS * BINWIDTH
HS = HEIGHTS_FLAT // NS

_mesh = plsc.VectorSubcoreMesh(core_axis_name="c", subcore_axis_name="s")


_G = 16


def _heights_body(s_ref, *refs):
    in_refs = refs[:2 * _G]
    o_ref = refs[2 * _G]
    for g in range(_G):
        b = in_refs[g][...]
        d = in_refs[_G + g][...]
        u = b + d
        m = jnp.max(u, axis=-1, keepdims=True)
        lse = jnp.log(jnp.sum(jnp.exp(u - m), axis=-1, keepdims=True)) + m
        o_ref[g] = (u - lse - math.log(BINSIZE))[0]


def _heights(baseline_weight, delta_logit_weight, regions_oi):
    def b_spec(g):
        return pl.BlockSpec((1, 1, BINWIDTH),
                            lambda i, s, g=g: (s[_G * i + g], 0, 0))

    def d_spec(g):
        return pl.BlockSpec((1, N_CLUSTERS, BINWIDTH),
                            lambda i, s, g=g: (s[_G * i + g], 0, 0))

    grid_spec = pltpu.PrefetchScalarGridSpec(
        num_scalar_prefetch=1,
        grid=(N_REGIONS_OI // _G,),
        in_specs=[b_spec(g) for g in range(_G)]
                 + [d_spec(g) for g in range(_G)],
        out_specs=pl.BlockSpec((_G, N_CLUSTERS, BINWIDTH),
                               lambda i, s: (i, 0, 0)),
    )
    return pl.pallas_call(
        _heights_body,
        grid_spec=grid_spec,
        out_shape=jax.ShapeDtypeStruct((N_REGIONS_OI, N_CLUSTERS, BINWIDTH),
                                       jnp.float32),
    )(regions_oi, *([baseline_weight.reshape(N_REGIONS, 1, BINWIDTH)] * _G),
      *([delta_logit_weight] * _G))


@functools.partial(
    pl.kernel,
    out_type=jax.ShapeDtypeStruct((2 * N_FRAG,), jnp.float32),
    mesh=_mesh,
    compiler_params=pltpu.CompilerParams(needs_layout_passes=False),
    scratch_types=[
        pltpu.VMEM((N_CELLS,), jnp.int32),
        pltpu.VMEM((16,), jnp.float32),
        pltpu.VMEM((16,), jnp.float32),
        pltpu.VMEM((TILE,), jnp.int32),
        pltpu.VMEM((TILE,), jnp.int32),
        pltpu.VMEM((TILE,), jnp.int32),
        pltpu.VMEM((TILE,), jnp.int32),
        pltpu.VMEM((TILE,), jnp.int32),
        pltpu.VMEM((TILE,), jnp.float32),
        pltpu.VMEM((2 * TILE,), jnp.float32),
        pltpu.VMEM_SHARED((HEIGHTS_FLAT,), jnp.float32),
        pltpu.SemaphoreType.DMA,
    ],
)
def _frag_phase(h_hbm, c0_hbm, c1_hbm, reg_hbm, cell_hbm, labels_hbm, cin_hbm,
                cout_hbm, out_hbm,
                labels_v, cin_v, cout_v, c0_v, c1_v, reg_v, cell_v, idx_v,
                gath_v, out_v, h_sp, sem):
    sid = lax.axis_index("s")
    cid = lax.axis_index("c")
    wid = sid * NC + cid

    pltpu.sync_copy(h_hbm.at[pl.ds(sid * HS, HS)], h_sp.at[pl.ds(sid * HS, HS)])
    pltpu.sync_copy(labels_hbm, labels_v)
    pltpu.sync_copy(cin_hbm, cin_v)
    pltpu.sync_copy(cout_hbm, cout_v)
    plsc.subcore_barrier()

    lanes = lax.iota(jnp.int32, 16)
    cin = cin_v[...]
    cout = cout_v[...]

    n_tiles = (NT - wid + NW - 1) // NW

    def tile_body(i, carry):
        t = wid + i * NW
        base = t * TILE
        pltpu.sync_copy(c0_hbm.at[pl.ds(base, TILE)], c0_v)
        pltpu.sync_copy(c1_hbm.at[pl.ds(base, TILE)], c1_v)
        pltpu.sync_copy(reg_hbm.at[pl.ds(base, TILE)], reg_v)
        pltpu.sync_copy(cell_hbm.at[pl.ds(base, TILE)], cell_v)

        def vec_body(j, c):
            o = j * 16
            pos = o + lanes
            c0 = c0_v[pl.ds(o, 16)]
            c1 = c1_v[pl.ds(o, 16)]
            cell = cell_v[pl.ds(o, 16)]
            reg = reg_v[pl.ds(o, 16)]
            clus = plsc.load_gather(labels_v, [cell])
            b0 = c0 // BINSIZE
            b1 = c1 // BINSIZE
            flat = reg * (N_CLUSTERS * BINWIDTH) + clus * BINWIDTH + b0
            idx_v[pl.ds(o, 16)] = flat
            lp1 = jnp.where(b0 == b1, cin, cout)
            plsc.store_scatter(out_v, [2 * pos + 1], lp1)
            return c

        lax.fori_loop(0, VECS, vec_body, 0)

        pltpu.async_copy(h_sp.at[idx_v], gath_v, sem).wait()

        def vec_body2(j, c):
            o = j * 16
            pos = o + lanes
            g = gath_v[pl.ds(o, 16)]
            plsc.store_scatter(out_v, [2 * pos], g)
            return c

        lax.fori_loop(0, VECS, vec_body2, 0)

        pltpu.sync_copy(out_v, out_hbm.at[pl.ds(2 * base, 2 * TILE)])
        return carry

    lax.fori_loop(0, n_tiles, tile_body, 0)


def kernel(baseline_weight, delta_logit_weight, inside, coordinates,
           local_region_ix, local_cell_ix, labels, regions_oi):
    heights = _heights(baseline_weight, delta_logit_weight, regions_oi)

    c0 = coordinates[:, 0]
    c1 = coordinates[:, 1]
    sig = jax.nn.sigmoid(inside)
    c_in = jnp.log(sig) - math.log(BINWIDTH)
    c_out = jnp.log(1.0 - sig) - math.log(WIDTH - BINWIDTH)
    cin16 = jnp.broadcast_to(c_in, (16,)).astype(jnp.float32)
    cout16 = jnp.broadcast_to(c_out, (16,)).astype(jnp.float32)

    out = _frag_phase(heights.reshape(HEIGHTS_FLAT), c0, c1,
                      local_region_ix, local_cell_ix, labels, cin16, cout16)
    return out.reshape(N_FRAG, 2)


# speedup vs baseline: 6.1716x; 1.0003x over previous
import functools
import math

import jax
import jax.numpy as jnp
from jax import lax
from jax.experimental import pallas as pl
from jax.experimental.pallas import tpu as pltpu
from jax.experimental.pallas import tpu_sc as plsc

BINSIZE = 200
WIDTH = 20000
BINWIDTH = 100
PADW = 128
N_REGIONS = 50000
N_CLUSTERS = 8
N_FRAG = 1000000
N_CELLS = 4096
N_REGIONS_OI = 1024

NC, NS = 2, 16
NW = NC * NS

TILE = 2000
NT = N_FRAG // TILE
VECS = TILE // 16

HEIGHTS_PAD = N_REGIONS_OI * N_CLUSTERS * PADW
HS = HEIGHTS_PAD // NS

_mesh = plsc.VectorSubcoreMesh(core_axis_name="c", subcore_axis_name="s")


_G = 16


def _heights_body(s_ref, *refs):
    in_refs = refs[:2 * _G]
    o_ref = refs[2 * _G]
    for g in range(_G):
        b = in_refs[g][...]
        d = in_refs[_G + g][...]
        u = b + d
        m = jnp.max(u, axis=-1, keepdims=True)
        lse = jnp.log(jnp.sum(jnp.exp(u - m), axis=-1, keepdims=True)) + m
        h = u - lse - math.log(BINSIZE)
        o_ref[g, :, :BINWIDTH] = h[0]


def _heights(baseline_weight, delta_logit_weight, regions_oi):
    def b_spec(g):
        return pl.BlockSpec((1, 1, BINWIDTH),
                            lambda i, s, g=g: (s[_G * i + g], 0, 0))

    def d_spec(g):
        return pl.BlockSpec((1, N_CLUSTERS, BINWIDTH),
                            lambda i, s, g=g: (s[_G * i + g], 0, 0))

    grid_spec = pltpu.PrefetchScalarGridSpec(
        num_scalar_prefetch=1,
        grid=(N_REGIONS_OI // _G,),
        in_specs=[b_spec(g) for g in range(_G)]
                 + [d_spec(g) for g in range(_G)],
        out_specs=pl.BlockSpec((_G, N_CLUSTERS, PADW),
                               lambda i, s: (i, 0, 0)),
    )
    return pl.pallas_call(
        _heights_body,
        grid_spec=grid_spec,
        out_shape=jax.ShapeDtypeStruct((N_REGIONS_OI, N_CLUSTERS, PADW),
                                       jnp.float32),
    )(regions_oi, *([baseline_weight.reshape(N_REGIONS, 1, BINWIDTH)] * _G),
      *([delta_logit_weight] * _G))


@functools.partial(
    pl.kernel,
    out_type=jax.ShapeDtypeStruct((2 * N_FRAG,), jnp.float32),
    mesh=_mesh,
    compiler_params=pltpu.CompilerParams(needs_layout_passes=False),
    scratch_types=[
        pltpu.VMEM((N_CELLS,), jnp.int32),
        pltpu.VMEM((16,), jnp.float32),
        pltpu.VMEM((16,), jnp.float32),
        pltpu.VMEM((TILE,), jnp.int32),
        pltpu.VMEM((TILE,), jnp.int32),
        pltpu.VMEM((TILE,), jnp.int32),
        pltpu.VMEM((TILE,), jnp.int32),
        pltpu.VMEM((TILE,), jnp.int32),
        pltpu.VMEM((TILE,), jnp.float32),
        pltpu.VMEM((2 * TILE,), jnp.float32),
        pltpu.VMEM_SHARED((HEIGHTS_PAD,), jnp.float32),
        pltpu.SemaphoreType.DMA,
    ],
)
def _frag_phase(h_hbm, c0_hbm, c1_hbm, reg_hbm, cell_hbm, labels_hbm, cin_hbm,
                cout_hbm, out_hbm,
                labels_v, cin_v, cout_v, c0_v, c1_v, reg_v, cell_v, idx_v,
                gath_v, out_v, h_sp, sem):
    sid = lax.axis_index("s")
    cid = lax.axis_index("c")
    wid = sid * NC + cid

    pltpu.sync_copy(h_hbm.at[pl.ds(sid * HS, HS)], h_sp.at[pl.ds(sid * HS, HS)])
    pltpu.sync_copy(labels_hbm, labels_v)
    pltpu.sync_copy(cin_hbm, cin_v)
    pltpu.sync_copy(cout_hbm, cout_v)
    plsc.subcore_barrier()

    lanes = lax.iota(jnp.int32, 16)
    cin = cin_v[...]
    cout = cout_v[...]

    n_tiles = (NT - wid + NW - 1) // NW

    def tile_body(i, carry):
        t = wid + i * NW
        base = t * TILE
        pltpu.sync_copy(c0_hbm.at[pl.ds(base, TILE)], c0_v)
        pltpu.sync_copy(c1_hbm.at[pl.ds(base, TILE)], c1_v)
        pltpu.sync_copy(reg_hbm.at[pl.ds(base, TILE)], reg_v)
        pltpu.sync_copy(cell_hbm.at[pl.ds(base, TILE)], cell_v)

        def vec_body(j, c):
            o = j * 16
            pos = o + lanes
            c0 = c0_v[pl.ds(o, 16)]
            c1 = c1_v[pl.ds(o, 16)]
            cell = cell_v[pl.ds(o, 16)]
            reg = reg_v[pl.ds(o, 16)]
            clus = plsc.load_gather(labels_v, [cell])
            b0 = c0 // BINSIZE
            b1 = c1 // BINSIZE
            flat = reg * (N_CLUSTERS * PADW) + clus * PADW + b0
            idx_v[pl.ds(o, 16)] = flat
            lp1 = jnp.where(b0 == b1, cin, cout)
            plsc.store_scatter(out_v, [2 * pos + 1], lp1)
            return c

        lax.fori_loop(0, VECS, vec_body, 0)

        pltpu.async_copy(h_sp.at[idx_v], gath_v, sem).wait()

        def vec_body2(j, c):
            o = j * 16
            pos = o + lanes
            g = gath_v[pl.ds(o, 16)]
            plsc.store_scatter(out_v, [2 * pos], g)
            return c

        lax.fori_loop(0, VECS, vec_body2, 0)

        pltpu.sync_copy(out_v, out_hbm.at[pl.ds(2 * base, 2 * TILE)])
        return carry

    lax.fori_loop(0, n_tiles, tile_body, 0)


def kernel(baseline_weight, delta_logit_weight, inside, coordinates,
           local_region_ix, local_cell_ix, labels, regions_oi):
    heights = _heights(baseline_weight, delta_logit_weight, regions_oi)

    c0 = coordinates[:, 0]
    c1 = coordinates[:, 1]
    sig = jax.nn.sigmoid(inside)
    c_in = jnp.log(sig) - math.log(BINWIDTH)
    c_out = jnp.log(1.0 - sig) - math.log(WIDTH - BINWIDTH)
    cin16 = jnp.broadcast_to(c_in, (16,)).astype(jnp.float32)
    cout16 = jnp.broadcast_to(c_out, (16,)).astype(jnp.float32)

    out = _frag_phase(heights.reshape(HEIGHTS_PAD), c0, c1,
                      local_region_ix, local_cell_ix, labels, cin16, cout16)
    return out.reshape(N_FRAG, 2)
